# baseline (speedup 1.0000x reference)
import functools

import jax
import jax.numpy as jnp
from jax import lax
from jax.experimental import pallas as pl
from jax.experimental.pallas import tpu as pltpu
from jax.experimental.pallas import tpu_sc as plsc

_N = 100000
_D = 256
_NW = 32
_R = 40
_NCHUNKS = _N // _R
_NPAIRS = (_NCHUNKS // _NW + 2) // 2


def _rsqrt_newton(s):
    i = lax.bitcast_convert_type(s, jnp.int32)
    i = jnp.int32(0x5F3759DF) - (i >> 1)
    y = lax.bitcast_convert_type(i, jnp.float32)
    for _ in range(3):
        y = y * (1.5 - 0.5 * s * y * y)
    return y


_GDN = lax.GatherDimensionNumbers(
    offset_dims=(), collapsed_slice_dims=(0,), start_index_map=(0,))


def _lane_shuffle(v, idx):
    return lax.gather(v, idx[:, None], dimension_numbers=_GDN,
                      slice_sizes=(1,),
                      mode=lax.GatherScatterMode.PROMISE_IN_BOUNDS)


def _allsum(v):
    lanes = lax.iota(jnp.int32, 16)
    for k in (8, 4, 2, 1):
        v = v + _lane_shuffle(v, lanes ^ k)
    return v


def _allmax(v):
    lanes = lax.iota(jnp.int32, 16)
    for k in (8, 4, 2, 1):
        v = jnp.maximum(v, _lane_shuffle(v, lanes ^ k))
    return v


def _sc_body(w_hbm, e0_hbm, e1_hbm, e2_hbm, out_hbm,
             wv, b0, b1, b2, bo, sin0, sin1, sout0, sout1):
    wid = lax.axis_index("s") * 2 + lax.axis_index("c")
    nmine = (_NCHUNKS - wid + _NW - 1) // _NW

    pltpu.sync_copy(w_hbm, wv)
    lanes = lax.iota(jnp.int32, 16)
    valid = lanes < 3
    w = jnp.where(valid, wv[:], -1e30)
    e = jnp.exp(w - _allmax(w))
    e = jnp.where(valid, e, 0.0)
    wn = e / _allsum(e)
    wms = [_allsum(jnp.where(lanes == m, wn, 0.0)) for m in range(3)]

    sin = (sin0, sin1)
    sout = (sout0, sout1)

    def row0_of(ci):
        return (wid + ci * _NW) * _R

    def in_copies(ci, slot):
        r0 = row0_of(ci)
        return (
            pltpu.make_async_copy(e0_hbm.at[pl.ds(r0, _R)], b0.at[slot], sin[slot]),
            pltpu.make_async_copy(e1_hbm.at[pl.ds(r0, _R)], b1.at[slot], sin[slot]),
            pltpu.make_async_copy(e2_hbm.at[pl.ds(r0, _R)], b2.at[slot], sin[slot]),
        )

    def out_copy(ci, slot):
        return pltpu.make_async_copy(
            bo.at[slot], out_hbm.at[pl.ds(row0_of(ci), _R)], sout[slot])

    def start_in(ci, slot):
        for c in in_copies(ci, slot):
            c.start()

    start_in(0, 0)

    def compute_chunk(slot):
        def row_body(r, _):
            for m, (inb, col) in enumerate(((b0, 0), (b1, _D), (b2, 2 * _D))):
                xs = [inb[slot, r, pl.ds(j * 16, 16)] for j in range(_D // 16)]
                acc = xs[0] * xs[0]
                for x in xs[1:]:
                    acc = acc + x * x
                s = _allsum(acc)
                norm = s * _rsqrt_newton(s)
                y = wms[m] / jnp.maximum(norm, 1e-12)
                for j, x in enumerate(xs):
                    bo[slot, r, pl.ds(col + j * 16, 16)] = x * y
            return 0
        lax.fori_loop(0, _R, row_body, 0)

    def pair_body(k, _):
        for slot in (0, 1):
            ci = 2 * k + slot

            @pl.when(ci < nmine)
            def _():
                for c in in_copies(ci, slot):
                    c.wait()

            @pl.when(ci + 1 < nmine)
            def _():
                start_in(ci + 1, slot ^ 1)

            @pl.when(jnp.logical_and(ci >= 2, ci < nmine))
            def _():
                out_copy(ci - 2, slot).wait()

            @pl.when(ci < nmine)
            def _():
                compute_chunk(slot)
                out_copy(ci, slot).start()
        return 0

    lax.fori_loop(0, _NPAIRS, pair_body, 0)

    out_copy(0, 0).wait()
    out_copy(0, 1).wait()


def kernel(emb0, emb1, emb2, weight):
    n, d = emb0.shape
    wpad = jnp.pad(weight.reshape(3), (0, 13))
    sc_call = functools.partial(
        pl.kernel,
        out_type=jax.ShapeDtypeStruct((n, 3 * d), emb0.dtype),
        mesh=plsc.VectorSubcoreMesh(core_axis_name="c", subcore_axis_name="s"),
        scratch_types=[
            pltpu.VMEM((16,), jnp.float32),
            pltpu.VMEM((2, _R, _D), jnp.float32),
            pltpu.VMEM((2, _R, _D), jnp.float32),
            pltpu.VMEM((2, _R, _D), jnp.float32),
            pltpu.VMEM((2, _R, 3 * _D), jnp.float32),
            pltpu.SemaphoreType.DMA,
            pltpu.SemaphoreType.DMA,
            pltpu.SemaphoreType.DMA,
            pltpu.SemaphoreType.DMA,
        ],
    )(_sc_body)
    return sc_call(wpad, emb0, emb1, emb2)


# speedup vs baseline: 3.2888x; 3.2888x over previous
import jax
import jax.numpy as jnp
from jax.experimental import pallas as pl

_N = 100000
_D = 256
_BLOCK = 4000


def _fuse_block(w_ref, e0_ref, e1_ref, e2_ref, out_ref):
    w = w_ref[:]
    e = jnp.exp(w - jnp.max(w))
    wn = e / jnp.sum(e)
    for i, ref in enumerate((e0_ref, e1_ref, e2_ref)):
        x = ref[:]
        n = jnp.sqrt(jnp.sum(x * x, axis=1, keepdims=True))
        out_ref[:, i * _D:(i + 1) * _D] = x / jnp.maximum(n, 1e-12) * wn[i]


def kernel(emb0, emb1, emb2, weight):
    n, d = emb0.shape
    grid = (n // _BLOCK,)
    emb_spec = pl.BlockSpec((_BLOCK, d), lambda i: (i, 0))
    return pl.pallas_call(
        _fuse_block,
        grid=grid,
        in_specs=[
            pl.BlockSpec((3, 1), lambda i: (0, 0)),
            emb_spec, emb_spec, emb_spec,
        ],
        out_specs=pl.BlockSpec((_BLOCK, 3 * d), lambda i: (i, 0)),
        out_shape=jax.ShapeDtypeStruct((n, 3 * d), emb0.dtype),
    )(weight, emb0, emb1, emb2)
